# baseline (speedup 1.0000x reference)
import functools

import jax
import jax.numpy as jnp
from jax import lax
from jax.experimental import pallas as pl
from jax.experimental.pallas import tpu as pltpu
from jax.experimental.pallas import tpu_sc as plsc

N = 10000
D = 128
E = 320000
E2 = 320000

NC = 2
NS = 16
NW = NC * NS
L = 16

C = 80
CHUNKS = E // C
CPW = CHUNKS // NW
RPT = N // NS

_mesh = functools.partial(
    plsc.VectorSubcoreMesh, core_axis_name="c", subcore_axis_name="s",
    num_cores=NC, num_subcores=NS)

_SC_PARAMS = pltpu.CompilerParams(use_tc_tiling_on_sc=False,
                                  needs_layout_passes=False)


def _tree_sum(vs):
    while len(vs) > 1:
        vs = [a + b for a, b in zip(vs[0::2], vs[1::2])]
    return vs[0]


def _deg_body(dst2d, zeros_hbm, ones_hbm, deg_out, idx_v, ones_v, deg_sh):
    c = lax.axis_index("c")
    s = lax.axis_index("s")
    w = c * NS + s
    pltpu.sync_copy(ones_hbm, ones_v)
    pltpu.sync_copy(dst2d.at[pl.ds(w * CPW, CPW)], idx_v)
    pltpu.sync_copy(zeros_hbm.at[pl.ds(s * RPT, RPT)],
                    deg_sh.at[pl.ds(s * RPT, RPT)])
    plsc.subcore_barrier()

    def body(j, carry):
        pltpu.sync_copy(ones_v, deg_sh.at[idx_v.at[j]], add=True)
        return carry

    lax.fori_loop(0, CPW, body, 0)
    plsc.subcore_barrier()
    pltpu.sync_copy(deg_sh.at[pl.ds(s * RPT, RPT)],
                    deg_out.at[c, pl.ds(s * RPT, RPT)])


def _sc_degree(dst2d, zeros16, ones16):
    k = pl.kernel(
        _deg_body,
        out_type=jax.ShapeDtypeStruct((NC, N, L), jnp.float32),
        mesh=_mesh(),
        compiler_params=_SC_PARAMS,
        scratch_types=[
            pltpu.VMEM((CPW, C), jnp.int32),
            pltpu.VMEM((C, L), jnp.float32),
            pltpu.VMEM_SHARED((N, L), jnp.float32),
        ],
    )
    return k(dst2d, zeros16, ones16)


ECH = 8000
NCH = E // ECH
FPT = D // NW


def _agg_body(gT_hbm, src1d, dst1d, accT,
              g_loc, acc_loc, sv0, dv0, sv1, dv1, se0, se1):
    c = lax.axis_index("c")
    s = lax.axis_index("s")
    w = c * NS + s
    pltpu.sync_copy(gT_hbm.at[pl.ds(FPT * w, FPT)], g_loc)
    pltpu.sync_copy(gT_hbm.at[pl.ds(FPT * w, FPT)], acc_loc)
    bufs = [(sv0, dv0, se0), (sv1, dv1, se1)]

    def issue(ch, sv, dv, sem):
        pltpu.async_copy(src1d.at[pl.ds(ch * ECH, ECH)], sv, sem)
        pltpu.async_copy(dst1d.at[pl.ds(ch * ECH, ECH)], dv, sem)

    def wait(sv, dv, sem):
        pltpu.make_async_copy(src1d.at[pl.ds(0, ECH)], sv, sem).wait()
        pltpu.make_async_copy(src1d.at[pl.ds(0, ECH)], dv, sem).wait()

    def compute(sv_ref, dv_ref):
        def gbody(gi, carry):
            svec = sv_ref[pl.ds(gi * L, L)]
            dvec = dv_ref[pl.ds(gi * L, L)]
            for f in range(FPT):
                val = plsc.load_gather(g_loc.at[f], [svec])
                plsc.addupdate_scatter(acc_loc.at[f], [dvec], val)
            return carry

        lax.fori_loop(0, ECH // L, gbody, 0, unroll=4)

    issue(0, *bufs[0])
    issue(1, *bufs[1])

    def body(ch2, carry):
        for b in range(2):
            ch = 2 * ch2 + b
            sv, dv, sem = bufs[b]
            wait(sv, dv, sem)
            compute(sv, dv)

            @pl.when(ch + 2 < NCH)
            def _():
                issue(ch + 2, sv, dv, sem)

        return carry

    lax.fori_loop(0, NCH // 2, body, 0)
    pltpu.sync_copy(acc_loc, accT.at[pl.ds(FPT * w, FPT)])


def _sc_aggregate(gT, src1d, dst1d):
    k = pl.kernel(
        _agg_body,
        out_type=jax.ShapeDtypeStruct((D, N), jnp.float32),
        mesh=_mesh(),
        compiler_params=_SC_PARAMS,
        scratch_types=[
            pltpu.VMEM((FPT, N), jnp.float32),
            pltpu.VMEM((FPT, N), jnp.float32),
            pltpu.VMEM((ECH,), jnp.int32),
            pltpu.VMEM((ECH,), jnp.int32),
            pltpu.VMEM((ECH,), jnp.int32),
            pltpu.VMEM((ECH,), jnp.int32),
            pltpu.SemaphoreType.DMA,
            pltpu.SemaphoreType.DMA,
        ],
    )
    return k(gT, src1d, dst1d)


DC = 64
DE2 = 321536
DCHUNKS = DE2 // DC
DCPW = DCHUNKS // NW
NBUF = 4


def _dec_body(z_hbm, ab2d, out, idx_v, r0, r1, r2, r3, sc_v, s0, s1, s2, s3):
    c = lax.axis_index("c")
    s = lax.axis_index("s")
    w = c * NS + s
    pltpu.sync_copy(ab2d.at[pl.ds(w * DCPW, DCPW)], idx_v)
    bufs = [(r0, s0), (r1, s1), (r2, s2), (r3, s3)]

    def issue(j, r, sem):
        pltpu.async_copy(z_hbm.at[idx_v.at[j]], r, sem)

    def wait(r, sem):
        pltpu.make_async_copy(z_hbm.at[pl.ds(0, 2 * DC)], r, sem).wait()

    lane = lax.iota(jnp.int32, L)
    perms = [lane ^ d for d in (8, 4, 2, 1)]

    def compute(j, r):
        def gbody(g0, carry):
            terms = []
            for e in range(L):
                row = g0 * L + e
                prods = []
                for k in range(D // (2 * L)):
                    va = r[row, pl.ds(k * 2 * L, 2 * L)]
                    vb = r[row + DC, pl.ds(k * 2 * L, 2 * L)]
                    a0, a1 = plsc.unpack(va, format=plsc.PackFormat.INTERLEAVED)
                    b0, b1 = plsc.unpack(vb, format=plsc.PackFormat.INTERLEAVED)
                    prods += [a0 * b0, a1 * b1]
                v = _tree_sum(prods)
                for p_ in perms:
                    v = v + v[p_]
                terms.append(jnp.where(lane == e, v, 0.0))
            sc_v[j, pl.ds(g0 * L, L)] = _tree_sum(terms)
            return carry

        lax.fori_loop(0, DC // L, gbody, 0)

    for b in range(NBUF):
        issue(b, *bufs[b])

    def body(j4, carry):
        for b in range(NBUF):
            j = NBUF * j4 + b

            @pl.when(j < DCPW)
            def _():
                wait(*bufs[b])
                compute(j, bufs[b][0])

                @pl.when(j + NBUF < DCPW)
                def _():
                    issue(j + NBUF, *bufs[b])

        return carry

    lax.fori_loop(0, (DCPW + NBUF - 1) // NBUF, body, 0)
    pltpu.sync_copy(sc_v, out.at[pl.ds(w * DCPW, DCPW)])


def _sc_decode(z, ab2d):
    k = pl.kernel(
        _dec_body,
        out_type=jax.ShapeDtypeStruct((DCHUNKS, DC), jnp.float32),
        mesh=_mesh(),
        compiler_params=_SC_PARAMS,
        scratch_types=[
            pltpu.VMEM((DCPW, 2 * DC), jnp.int32),
            pltpu.VMEM((2 * DC, D), jnp.bfloat16),
            pltpu.VMEM((2 * DC, D), jnp.bfloat16),
            pltpu.VMEM((2 * DC, D), jnp.bfloat16),
            pltpu.VMEM((2 * DC, D), jnp.bfloat16),
            pltpu.VMEM((DCPW, DC), jnp.float32),
            pltpu.SemaphoreType.DMA,
            pltpu.SemaphoreType.DMA,
            pltpu.SemaphoreType.DMA,
            pltpu.SemaphoreType.DMA,
        ],
    )
    return k(z, ab2d)


BLK = N
GRID = 1


def _tc_a_body(x_ref, w1_ref, deg_ref, gT_ref, dinv_ref):
    deg = deg_ref[0] + deg_ref[1]
    dinv = lax.rsqrt(deg + 1.0)[:, :1]
    h = jnp.dot(x_ref[...], w1_ref[...],
                preferred_element_type=jnp.float32,
                precision=lax.Precision.HIGHEST)
    gT_ref[...] = jnp.transpose(dinv * h, (1, 0))
    dinv_ref[...] = jnp.transpose(dinv, (1, 0))


def _tc_a(x, W1, deg16):
    return pl.pallas_call(
        _tc_a_body,
        grid=(GRID,),
        in_specs=[
            pl.BlockSpec((BLK, D), lambda i: (i, 0)),
            pl.BlockSpec((D, D), lambda i: (0, 0)),
            pl.BlockSpec((NC, BLK, L), lambda i: (0, i, 0)),
        ],
        out_specs=[
            pl.BlockSpec((D, BLK), lambda i: (0, i)),
            pl.BlockSpec((1, BLK), lambda i: (0, i)),
        ],
        out_shape=[
            jax.ShapeDtypeStruct((D, N), jnp.float32),
            jax.ShapeDtypeStruct((1, N), jnp.float32),
        ],
    )(x, W1, deg16)


def _tc_b_body(accT_ref, dinv_ref, b1_ref, w2_ref, g2T_ref):
    z1 = jnp.maximum(dinv_ref[...] * accT_ref[...] + b1_ref[...], 0.0)
    h2 = lax.dot_general(w2_ref[...], z1, (((0,), (0,)), ((), ())),
                         preferred_element_type=jnp.float32,
                         precision=lax.Precision.HIGHEST)
    g2T_ref[...] = dinv_ref[...] * h2


def _tc_b(acc1T, dinvr, b1, W2):
    return pl.pallas_call(
        _tc_b_body,
        grid=(GRID,),
        in_specs=[
            pl.BlockSpec((D, BLK), lambda i: (0, i)),
            pl.BlockSpec((1, BLK), lambda i: (0, i)),
            pl.BlockSpec((D, 1), lambda i: (0, 0)),
            pl.BlockSpec((D, D), lambda i: (0, 0)),
        ],
        out_specs=pl.BlockSpec((D, BLK), lambda i: (0, i)),
        out_shape=jax.ShapeDtypeStruct((D, N), jnp.float32),
    )(acc1T, dinvr, b1, W2)


def _tc_c_body(accT_ref, dinv_ref, b2_ref, z_ref):
    zT = dinv_ref[...] * accT_ref[...] + b2_ref[...]
    z_ref[...] = jnp.transpose(zT, (1, 0)).astype(jnp.bfloat16)


def _tc_c(acc2T, dinvr, b2):
    return pl.pallas_call(
        _tc_c_body,
        grid=(GRID,),
        in_specs=[
            pl.BlockSpec((D, BLK), lambda i: (0, i)),
            pl.BlockSpec((1, BLK), lambda i: (0, i)),
            pl.BlockSpec((D, 1), lambda i: (0, 0)),
        ],
        out_specs=pl.BlockSpec((BLK, D), lambda i: (i, 0)),
        out_shape=jax.ShapeDtypeStruct((N, D), jnp.bfloat16),
    )(acc2T, dinvr, b2)


def kernel(x, edge_index, pos_edge_index, neg_edge_index, W1, b1, W2, b2):
    src2d = edge_index[0].reshape(CHUNKS, C)
    dst2d = edge_index[1].reshape(CHUNKS, C)
    dec = jnp.concatenate([pos_edge_index, neg_edge_index], axis=1)
    pad = jnp.zeros((2, DE2 - E2), jnp.int32)
    dec = jnp.concatenate([dec, pad], axis=1)
    ab2d = jnp.concatenate([dec[0].reshape(DCHUNKS, DC),
                            dec[1].reshape(DCHUNKS, DC)], axis=1)
    zeros16 = jnp.zeros((N, L), jnp.float32)
    ones16 = jnp.ones((C, L), jnp.float32)

    deg16 = _sc_degree(dst2d, zeros16, ones16)
    g1T, dinvr = _tc_a(x, W1, deg16)
    acc1T = _sc_aggregate(g1T, edge_index[0], edge_index[1])
    g2T = _tc_b(acc1T, dinvr, b1.reshape(D, 1), W2)
    acc2T = _sc_aggregate(g2T, edge_index[0], edge_index[1])
    z = _tc_c(acc2T, dinvr, b2.reshape(D, 1))
    scores = _sc_decode(z, ab2d).reshape(DE2)
    return scores[:E2 // 2], scores[E2 // 2:E2]


# speedup vs baseline: 1.9445x; 1.9445x over previous
import functools

import jax
import jax.numpy as jnp
from jax import lax
from jax.experimental import pallas as pl
from jax.experimental.pallas import tpu as pltpu
from jax.experimental.pallas import tpu_sc as plsc

N = 10000
D = 128
E = 320000
E2 = 320000

NC = 2
NS = 16
NW = NC * NS
L = 16

C = 80
CHUNKS = E // C
CPW = CHUNKS // NW
RPT = N // NS

_mesh = functools.partial(
    plsc.VectorSubcoreMesh, core_axis_name="c", subcore_axis_name="s",
    num_cores=NC, num_subcores=NS)

_SC_PARAMS = pltpu.CompilerParams(use_tc_tiling_on_sc=False,
                                  needs_layout_passes=False)


def _tree_sum(vs):
    while len(vs) > 1:
        vs = [a + b for a, b in zip(vs[0::2], vs[1::2])]
    return vs[0]


def _deg_body(dst2d, zeros_hbm, ones_hbm, deg_out, idx_v, ones_v, deg_sh):
    c = lax.axis_index("c")
    s = lax.axis_index("s")
    w = c * NS + s
    pltpu.sync_copy(ones_hbm, ones_v)
    pltpu.sync_copy(dst2d.at[pl.ds(w * CPW, CPW)], idx_v)
    pltpu.sync_copy(zeros_hbm.at[pl.ds(s * RPT, RPT)],
                    deg_sh.at[pl.ds(s * RPT, RPT)])
    plsc.subcore_barrier()

    def body(j, carry):
        pltpu.sync_copy(ones_v, deg_sh.at[idx_v.at[j]], add=True)
        return carry

    lax.fori_loop(0, CPW, body, 0)
    plsc.subcore_barrier()
    pltpu.sync_copy(deg_sh.at[pl.ds(s * RPT, RPT)],
                    deg_out.at[c, pl.ds(s * RPT, RPT)])


def _sc_degree(dst2d, zeros16, ones16):
    k = pl.kernel(
        _deg_body,
        out_type=jax.ShapeDtypeStruct((NC, N, L), jnp.float32),
        mesh=_mesh(),
        compiler_params=_SC_PARAMS,
        scratch_types=[
            pltpu.VMEM((CPW, C), jnp.int32),
            pltpu.VMEM((C, L), jnp.float32),
            pltpu.VMEM_SHARED((N, L), jnp.float32),
        ],
    )
    return k(dst2d, zeros16, ones16)


def _agg_body(g_hbm, src2d, dst2d, out,
              sidx_v, didx_v, r0, r1, acc_sh, gs0, gs1, ss0, ss1):
    c = lax.axis_index("c")
    s = lax.axis_index("s")
    w = c * NS + s
    pltpu.sync_copy(src2d.at[pl.ds(w * CPW, CPW)], sidx_v)
    pltpu.sync_copy(dst2d.at[pl.ds(w * CPW, CPW)], didx_v)
    pltpu.sync_copy(g_hbm.at[pl.ds(s * RPT, RPT)],
                    acc_sh.at[pl.ds(s * RPT, RPT)])
    plsc.subcore_barrier()

    def g_issue(j, r, sem):
        pltpu.async_copy(g_hbm.at[sidx_v.at[j]], r, sem)

    def g_wait(r, sem):
        pltpu.make_async_copy(g_hbm.at[pl.ds(0, C)], r, sem).wait()

    def s_issue(j, r, sem):
        pltpu.async_copy(r, acc_sh.at[didx_v.at[j]], sem, add=True)

    def s_wait(r, sem):
        pltpu.make_async_copy(r, acc_sh.at[pl.ds(0, C)], sem).wait()

    g_issue(0, r0, gs0)
    g_issue(1, r1, gs1)

    def body(j2, carry):
        j = 2 * j2
        g_wait(r0, gs0)
        s_issue(j, r0, ss0)
        s_wait(r0, ss0)

        @pl.when(j + 2 < CPW)
        def _():
            g_issue(j + 2, r0, gs0)

        @pl.when(j + 1 < CPW)
        def _():
            g_wait(r1, gs1)
            s_issue(j + 1, r1, ss1)
            s_wait(r1, ss1)

            @pl.when(j + 3 < CPW)
            def _():
                g_issue(j + 3, r1, gs1)

        return carry

    lax.fori_loop(0, (CPW + 1) // 2, body, 0)
    plsc.subcore_barrier()
    pltpu.sync_copy(acc_sh.at[pl.ds(s * RPT, RPT)],
                    out.at[c, pl.ds(s * RPT, RPT)])


def _sc_aggregate(g, src2d, dst2d):
    k = pl.kernel(
        _agg_body,
        out_type=jax.ShapeDtypeStruct((NC, N, D), jnp.float32),
        mesh=_mesh(),
        compiler_params=_SC_PARAMS,
        scratch_types=[
            pltpu.VMEM((CPW, C), jnp.int32),
            pltpu.VMEM((CPW, C), jnp.int32),
            pltpu.VMEM((C, D), jnp.float32),
            pltpu.VMEM((C, D), jnp.float32),
            pltpu.VMEM_SHARED((N, D), jnp.float32),
            pltpu.SemaphoreType.DMA,
            pltpu.SemaphoreType.DMA,
            pltpu.SemaphoreType.DMA,
            pltpu.SemaphoreType.DMA,
        ],
    )
    return k(g, src2d, dst2d)


DC = 64
DE2 = 323584
DCHUNKS = DE2 // DC
DCPW = DCHUNKS // NW
SPW = DCPW // 2
NBUF = 4


def _dec_body(z_hbm, ab2d, out, idx_v, r0, r1, r2, r3, sc_v, s0, s1, s2, s3):
    c = lax.axis_index("c")
    s = lax.axis_index("s")
    w = c * NS + s
    pltpu.sync_copy(ab2d.at[pl.ds(w * DCPW, DCPW)], idx_v)
    bufs = [(r0, s0), (r1, s1), (r2, s2), (r3, s3)]

    def issue(t, r, sem):
        pltpu.async_copy(z_hbm.at[idx_v.at[2 * t]], r.at[pl.ds(0, 2 * DC)],
                         sem)
        pltpu.async_copy(z_hbm.at[idx_v.at[2 * t + 1]],
                         r.at[pl.ds(2 * DC, 2 * DC)], sem)

    def wait(r, sem):
        pltpu.make_async_copy(z_hbm.at[pl.ds(0, 2 * DC)],
                              r.at[pl.ds(0, 2 * DC)], sem).wait()
        pltpu.make_async_copy(z_hbm.at[pl.ds(0, 2 * DC)],
                              r.at[pl.ds(2 * DC, 2 * DC)], sem).wait()

    lane = lax.iota(jnp.int32, L)
    perms = [lane ^ d for d in (8, 4, 2, 1)]

    def compute(t, r):
        def gbody(g0, carry):
            half = g0 >> 2
            base = half * (2 * DC) + (g0 & 3) * L
            terms = []
            for e in range(L):
                row = base + e
                prods = []
                for k in range(D // (2 * L)):
                    va = r[row, pl.ds(k * 2 * L, 2 * L)]
                    vb = r[row + DC, pl.ds(k * 2 * L, 2 * L)]
                    a0, a1 = plsc.unpack(va, format=plsc.PackFormat.INTERLEAVED)
                    b0, b1 = plsc.unpack(vb, format=plsc.PackFormat.INTERLEAVED)
                    prods += [a0 * b0, a1 * b1]
                v = _tree_sum(prods)
                for p_ in perms:
                    v = v + v[p_]
                terms.append(jnp.where(lane == e, v, 0.0))
            sc_v[pl.ds(t * (2 * DC) + g0 * L, L)] = _tree_sum(terms)
            return carry

        lax.fori_loop(0, 2 * DC // L, gbody, 0)

    for b in range(NBUF):
        issue(b, *bufs[b])

    def body(t4, carry):
        for b in range(NBUF):
            t = NBUF * t4 + b

            @pl.when(t < SPW)
            def _():
                wait(*bufs[b])
                compute(t, bufs[b][0])

                @pl.when(t + NBUF < SPW)
                def _():
                    issue(t + NBUF, *bufs[b])

        return carry

    lax.fori_loop(0, (SPW + NBUF - 1) // NBUF, body, 0)
    pltpu.sync_copy(sc_v, out.at[pl.ds(w * DCPW * DC, DCPW * DC)])


def _sc_decode(z, ab2d):
    k = pl.kernel(
        _dec_body,
        out_type=jax.ShapeDtypeStruct((DE2,), jnp.float32),
        mesh=_mesh(),
        compiler_params=_SC_PARAMS,
        scratch_types=[
            pltpu.VMEM((DCPW, 2 * DC), jnp.int32),
            pltpu.VMEM((4 * DC, D), jnp.bfloat16),
            pltpu.VMEM((4 * DC, D), jnp.bfloat16),
            pltpu.VMEM((4 * DC, D), jnp.bfloat16),
            pltpu.VMEM((4 * DC, D), jnp.bfloat16),
            pltpu.VMEM((DCPW * DC,), jnp.float32),
            pltpu.SemaphoreType.DMA,
            pltpu.SemaphoreType.DMA,
            pltpu.SemaphoreType.DMA,
            pltpu.SemaphoreType.DMA,
        ],
    )
    return k(z, ab2d)


BLK = 1000
GRID = N // BLK


def _tc_a_body(x_ref, w1_ref, deg_ref, g_ref):
    deg = deg_ref[0] + deg_ref[1]
    dinv = lax.rsqrt(deg + 1.0)[:, :1]
    h = jnp.dot(x_ref[...], w1_ref[...],
                preferred_element_type=jnp.float32,
                precision=lax.Precision.HIGHEST)
    g_ref[...] = dinv * h


def _tc_a(x, W1, deg16):
    return pl.pallas_call(
        _tc_a_body,
        grid=(GRID,),
        in_specs=[
            pl.BlockSpec((BLK, D), lambda i: (i, 0)),
            pl.BlockSpec((D, D), lambda i: (0, 0)),
            pl.BlockSpec((NC, BLK, L), lambda i: (0, i, 0)),
        ],
        out_specs=pl.BlockSpec((BLK, D), lambda i: (i, 0)),
        out_shape=jax.ShapeDtypeStruct((N, D), jnp.float32),
    )(x, W1, deg16)


def _tc_b_body(acc_ref, g1_ref, deg_ref, b1_ref, w2_ref, g2_ref):
    deg = deg_ref[0] + deg_ref[1]
    dinv = lax.rsqrt(deg + 1.0)[:, :1]
    z1 = dinv * (acc_ref[0] + acc_ref[1] - g1_ref[...]) + b1_ref[...]
    z1 = jnp.maximum(z1, 0.0)
    h = jnp.dot(z1, w2_ref[...],
                preferred_element_type=jnp.float32,
                precision=lax.Precision.HIGHEST)
    g2_ref[...] = dinv * h


def _tc_b(acc1, g1, deg16, b1, W2):
    return pl.pallas_call(
        _tc_b_body,
        grid=(GRID,),
        in_specs=[
            pl.BlockSpec((NC, BLK, D), lambda i: (0, i, 0)),
            pl.BlockSpec((BLK, D), lambda i: (i, 0)),
            pl.BlockSpec((NC, BLK, L), lambda i: (0, i, 0)),
            pl.BlockSpec((1, D), lambda i: (0, 0)),
            pl.BlockSpec((D, D), lambda i: (0, 0)),
        ],
        out_specs=pl.BlockSpec((BLK, D), lambda i: (i, 0)),
        out_shape=jax.ShapeDtypeStruct((N, D), jnp.float32),
    )(acc1, g1, deg16, b1, W2)


def _tc_c_body(acc_ref, g2_ref, deg_ref, b2_ref, z_ref):
    deg = deg_ref[0] + deg_ref[1]
    dinv = lax.rsqrt(deg + 1.0)[:, :1]
    z = dinv * (acc_ref[0] + acc_ref[1] - g2_ref[...]) + b2_ref[...]
    z_ref[...] = z.astype(jnp.bfloat16)


def _tc_c(acc2, g2, deg16, b2):
    return pl.pallas_call(
        _tc_c_body,
        grid=(GRID,),
        in_specs=[
            pl.BlockSpec((NC, BLK, D), lambda i: (0, i, 0)),
            pl.BlockSpec((BLK, D), lambda i: (i, 0)),
            pl.BlockSpec((NC, BLK, L), lambda i: (0, i, 0)),
            pl.BlockSpec((1, D), lambda i: (0, 0)),
        ],
        out_specs=pl.BlockSpec((BLK, D), lambda i: (i, 0)),
        out_shape=jax.ShapeDtypeStruct((N, D), jnp.bfloat16),
    )(acc2, g2, deg16, b2)


def kernel(x, edge_index, pos_edge_index, neg_edge_index, W1, b1, W2, b2):
    src2d = edge_index[0].reshape(CHUNKS, C)
    dst2d = edge_index[1].reshape(CHUNKS, C)
    dec = jnp.concatenate([pos_edge_index, neg_edge_index], axis=1)
    pad = jnp.zeros((2, DE2 - E2), jnp.int32)
    dec = jnp.concatenate([dec, pad], axis=1)
    ab2d = jnp.concatenate([dec[0].reshape(DCHUNKS, DC),
                            dec[1].reshape(DCHUNKS, DC)], axis=1)
    zeros16 = jnp.zeros((N, L), jnp.float32)
    ones16 = jnp.ones((C, L), jnp.float32)

    deg16 = _sc_degree(dst2d, zeros16, ones16)
    g1 = _tc_a(x, W1, deg16)
    acc1 = _sc_aggregate(g1, src2d, dst2d)
    g2 = _tc_b(acc1, g1, deg16, b1.reshape(1, D), W2)
    acc2 = _sc_aggregate(g2, src2d, dst2d)
    z = _tc_c(acc2, g2, deg16, b2.reshape(1, D))
    scores = _sc_decode(z, ab2d)
    return scores[:E2 // 2], scores[E2 // 2:E2]


# speedup vs baseline: 2.6269x; 1.3509x over previous
import functools

import jax
import jax.numpy as jnp
from jax import lax
from jax.experimental import pallas as pl
from jax.experimental.pallas import tpu as pltpu
from jax.experimental.pallas import tpu_sc as plsc

N = 10000
D = 128
E = 320000
E2 = 320000

NC = 2
NS = 16
NW = NC * NS
L = 16

C = 80
CHUNKS = E // C
CPW = CHUNKS // NW
RPT = N // NS

_mesh = functools.partial(
    plsc.VectorSubcoreMesh, core_axis_name="c", subcore_axis_name="s",
    num_cores=NC, num_subcores=NS)

_SC_PARAMS = pltpu.CompilerParams(use_tc_tiling_on_sc=False,
                                  needs_layout_passes=False)


def _tree_sum(vs):
    while len(vs) > 1:
        vs = [a + b for a, b in zip(vs[0::2], vs[1::2])]
    return vs[0]


def _deg_body(dst2d, zeros_hbm, ones_hbm, deg_out, idx_v, ones_v, deg_sh):
    c = lax.axis_index("c")
    s = lax.axis_index("s")
    w = c * NS + s
    pltpu.sync_copy(ones_hbm, ones_v)
    pltpu.sync_copy(dst2d.at[pl.ds(w * CPW, CPW)], idx_v)
    pltpu.sync_copy(zeros_hbm.at[pl.ds(s * RPT, RPT)],
                    deg_sh.at[pl.ds(s * RPT, RPT)])
    plsc.subcore_barrier()

    def body(j, carry):
        pltpu.sync_copy(ones_v, deg_sh.at[idx_v.at[j]], add=True)
        return carry

    lax.fori_loop(0, CPW, body, 0)
    plsc.subcore_barrier()
    pltpu.sync_copy(deg_sh.at[pl.ds(s * RPT, RPT)],
                    deg_out.at[c, pl.ds(s * RPT, RPT)])


def _sc_degree(dst2d, zeros16, ones16):
    k = pl.kernel(
        _deg_body,
        out_type=jax.ShapeDtypeStruct((NC, N, L), jnp.float32),
        mesh=_mesh(),
        compiler_params=_SC_PARAMS,
        scratch_types=[
            pltpu.VMEM((CPW, C), jnp.int32),
            pltpu.VMEM((C, L), jnp.float32),
            pltpu.VMEM_SHARED((N, L), jnp.float32),
        ],
    )
    return k(dst2d, zeros16, ones16)


def _agg_body(g_hbm, src2d, dst2d, out,
              sidx_v, didx_v, r0, r1, acc_sh, gs0, gs1, ss0, ss1):
    c = lax.axis_index("c")
    s = lax.axis_index("s")
    w = c * NS + s
    pltpu.sync_copy(src2d.at[pl.ds(w * CPW, CPW)], sidx_v)
    pltpu.sync_copy(dst2d.at[pl.ds(w * CPW, CPW)], didx_v)
    pltpu.sync_copy(g_hbm.at[pl.ds(s * RPT, RPT)],
                    acc_sh.at[pl.ds(s * RPT, RPT)])
    plsc.subcore_barrier()

    def g_issue(j, r, sem):
        pltpu.async_copy(g_hbm.at[sidx_v.at[j]], r, sem)

    def g_wait(r, sem):
        pltpu.make_async_copy(g_hbm.at[pl.ds(0, C)], r, sem).wait()

    def s_issue(j, r, sem):
        pltpu.async_copy(r, acc_sh.at[didx_v.at[j]], sem, add=True)

    def s_wait(r, sem):
        pltpu.make_async_copy(r, acc_sh.at[pl.ds(0, C)], sem).wait()

    g_issue(0, r0, gs0)
    g_issue(1, r1, gs1)

    def body(j2, carry):
        j = 2 * j2
        g_wait(r0, gs0)
        s_issue(j, r0, ss0)
        s_wait(r0, ss0)

        @pl.when(j + 2 < CPW)
        def _():
            g_issue(j + 2, r0, gs0)

        @pl.when(j + 1 < CPW)
        def _():
            g_wait(r1, gs1)
            s_issue(j + 1, r1, ss1)
            s_wait(r1, ss1)

            @pl.when(j + 3 < CPW)
            def _():
                g_issue(j + 3, r1, gs1)

        return carry

    lax.fori_loop(0, (CPW + 1) // 2, body, 0)
    plsc.subcore_barrier()
    pltpu.sync_copy(acc_sh.at[pl.ds(s * RPT, RPT)],
                    out.at[c, pl.ds(s * RPT, RPT)])


def _sc_aggregate(g, src2d, dst2d):
    k = pl.kernel(
        _agg_body,
        out_type=jax.ShapeDtypeStruct((NC, N, D), jnp.float32),
        mesh=_mesh(),
        compiler_params=_SC_PARAMS,
        scratch_types=[
            pltpu.VMEM((CPW, C), jnp.int32),
            pltpu.VMEM((CPW, C), jnp.int32),
            pltpu.VMEM((C, D), jnp.float32),
            pltpu.VMEM((C, D), jnp.float32),
            pltpu.VMEM_SHARED((N, D), jnp.float32),
            pltpu.SemaphoreType.DMA,
            pltpu.SemaphoreType.DMA,
            pltpu.SemaphoreType.DMA,
            pltpu.SemaphoreType.DMA,
        ],
    )
    return k(g, src2d, dst2d)


DC = 64
DE2 = 321536
DCHUNKS = DE2 // DC
DCPW = DCHUNKS // NW
NBUF = 4


def _dec_body(z_hbm, ab2d, out, idx_v, r0, r1, r2, r3, sc_v, z_sh,
              s0, s1, s2, s3):
    c = lax.axis_index("c")
    s = lax.axis_index("s")
    w = c * NS + s
    pltpu.sync_copy(ab2d.at[pl.ds(w * DCPW, DCPW)], idx_v)
    pltpu.sync_copy(z_hbm.at[pl.ds(s * RPT, RPT)],
                    z_sh.at[pl.ds(s * RPT, RPT)])
    plsc.subcore_barrier()
    bufs = [(r0, s0), (r1, s1), (r2, s2), (r3, s3)]

    def issue(j, r, sem):
        pltpu.async_copy(z_sh.at[idx_v.at[j]], r, sem)

    def wait(r, sem):
        pltpu.make_async_copy(z_hbm.at[pl.ds(0, 2 * DC)], r, sem).wait()

    lane = lax.iota(jnp.int32, L)
    perms = [lane ^ d for d in (8, 4, 2, 1)]

    def compute(j, r):
        def gbody(g0, carry):
            terms = []
            for e in range(L):
                row = g0 * L + e
                prods = []
                for k in range(D // (2 * L)):
                    va = r[row, pl.ds(k * 2 * L, 2 * L)]
                    vb = r[row + DC, pl.ds(k * 2 * L, 2 * L)]
                    a0, a1 = plsc.unpack(va, format=plsc.PackFormat.INTERLEAVED)
                    b0, b1 = plsc.unpack(vb, format=plsc.PackFormat.INTERLEAVED)
                    prods += [a0 * b0, a1 * b1]
                v = _tree_sum(prods)
                for p_ in perms:
                    v = v + v[p_]
                terms.append(jnp.where(lane == e, v, 0.0))
            sc_v[j, pl.ds(g0 * L, L)] = _tree_sum(terms)
            return carry

        lax.fori_loop(0, DC // L, gbody, 0)

    for b in range(NBUF):
        issue(b, *bufs[b])

    def body(j4, carry):
        for b in range(NBUF):
            j = NBUF * j4 + b

            @pl.when(j < DCPW)
            def _():
                wait(*bufs[b])
                compute(j, bufs[b][0])

                @pl.when(j + NBUF < DCPW)
                def _():
                    issue(j + NBUF, *bufs[b])

        return carry

    lax.fori_loop(0, (DCPW + NBUF - 1) // NBUF, body, 0)
    pltpu.sync_copy(sc_v, out.at[pl.ds(w * DCPW, DCPW)])


def _sc_decode(z, ab2d):
    k = pl.kernel(
        _dec_body,
        out_type=jax.ShapeDtypeStruct((DCHUNKS, DC), jnp.float32),
        mesh=_mesh(),
        compiler_params=_SC_PARAMS,
        scratch_types=[
            pltpu.VMEM((DCPW, 2 * DC), jnp.int32),
            pltpu.VMEM((2 * DC, D), jnp.bfloat16),
            pltpu.VMEM((2 * DC, D), jnp.bfloat16),
            pltpu.VMEM((2 * DC, D), jnp.bfloat16),
            pltpu.VMEM((2 * DC, D), jnp.bfloat16),
            pltpu.VMEM((DCPW, DC), jnp.float32),
            pltpu.VMEM_SHARED((N, D), jnp.bfloat16),
            pltpu.SemaphoreType.DMA,
            pltpu.SemaphoreType.DMA,
            pltpu.SemaphoreType.DMA,
            pltpu.SemaphoreType.DMA,
        ],
    )
    return k(z, ab2d)


BLK = 1000
GRID = N // BLK


def _tc_a_body(x_ref, w1_ref, deg_ref, g_ref):
    deg = deg_ref[0] + deg_ref[1]
    dinv = lax.rsqrt(deg + 1.0)[:, :1]
    h = jnp.dot(x_ref[...], w1_ref[...],
                preferred_element_type=jnp.float32,
                precision=lax.Precision.HIGHEST)
    g_ref[...] = dinv * h


def _tc_a(x, W1, deg16):
    return pl.pallas_call(
        _tc_a_body,
        grid=(GRID,),
        in_specs=[
            pl.BlockSpec((BLK, D), lambda i: (i, 0)),
            pl.BlockSpec((D, D), lambda i: (0, 0)),
            pl.BlockSpec((NC, BLK, L), lambda i: (0, i, 0)),
        ],
        out_specs=pl.BlockSpec((BLK, D), lambda i: (i, 0)),
        out_shape=jax.ShapeDtypeStruct((N, D), jnp.float32),
    )(x, W1, deg16)


def _tc_b_body(acc_ref, g1_ref, deg_ref, b1_ref, w2_ref, g2_ref):
    deg = deg_ref[0] + deg_ref[1]
    dinv = lax.rsqrt(deg + 1.0)[:, :1]
    z1 = dinv * (acc_ref[0] + acc_ref[1] - g1_ref[...]) + b1_ref[...]
    z1 = jnp.maximum(z1, 0.0)
    h = jnp.dot(z1, w2_ref[...],
                preferred_element_type=jnp.float32,
                precision=lax.Precision.HIGHEST)
    g2_ref[...] = dinv * h


def _tc_b(acc1, g1, deg16, b1, W2):
    return pl.pallas_call(
        _tc_b_body,
        grid=(GRID,),
        in_specs=[
            pl.BlockSpec((NC, BLK, D), lambda i: (0, i, 0)),
            pl.BlockSpec((BLK, D), lambda i: (i, 0)),
            pl.BlockSpec((NC, BLK, L), lambda i: (0, i, 0)),
            pl.BlockSpec((1, D), lambda i: (0, 0)),
            pl.BlockSpec((D, D), lambda i: (0, 0)),
        ],
        out_specs=pl.BlockSpec((BLK, D), lambda i: (i, 0)),
        out_shape=jax.ShapeDtypeStruct((N, D), jnp.float32),
    )(acc1, g1, deg16, b1, W2)


def _tc_c_body(acc_ref, g2_ref, deg_ref, b2_ref, z_ref):
    deg = deg_ref[0] + deg_ref[1]
    dinv = lax.rsqrt(deg + 1.0)[:, :1]
    z = dinv * (acc_ref[0] + acc_ref[1] - g2_ref[...]) + b2_ref[...]
    z_ref[...] = z.astype(jnp.bfloat16)


def _tc_c(acc2, g2, deg16, b2):
    return pl.pallas_call(
        _tc_c_body,
        grid=(GRID,),
        in_specs=[
            pl.BlockSpec((NC, BLK, D), lambda i: (0, i, 0)),
            pl.BlockSpec((BLK, D), lambda i: (i, 0)),
            pl.BlockSpec((NC, BLK, L), lambda i: (0, i, 0)),
            pl.BlockSpec((1, D), lambda i: (0, 0)),
        ],
        out_specs=pl.BlockSpec((BLK, D), lambda i: (i, 0)),
        out_shape=jax.ShapeDtypeStruct((N, D), jnp.bfloat16),
    )(acc2, g2, deg16, b2)


def kernel(x, edge_index, pos_edge_index, neg_edge_index, W1, b1, W2, b2):
    src2d = edge_index[0].reshape(CHUNKS, C)
    dst2d = edge_index[1].reshape(CHUNKS, C)
    dec = jnp.concatenate([pos_edge_index, neg_edge_index], axis=1)
    pad = jnp.zeros((2, DE2 - E2), jnp.int32)
    dec = jnp.concatenate([dec, pad], axis=1)
    ab2d = jnp.concatenate([dec[0].reshape(DCHUNKS, DC),
                            dec[1].reshape(DCHUNKS, DC)], axis=1)
    zeros16 = jnp.zeros((N, L), jnp.float32)
    ones16 = jnp.ones((C, L), jnp.float32)

    deg16 = _sc_degree(dst2d, zeros16, ones16)
    g1 = _tc_a(x, W1, deg16)
    acc1 = _sc_aggregate(g1, src2d, dst2d)
    g2 = _tc_b(acc1, g1, deg16, b1.reshape(1, D), W2)
    acc2 = _sc_aggregate(g2, src2d, dst2d)
    z = _tc_c(acc2, g2, deg16, b2.reshape(1, D))
    scores = _sc_decode(z, ab2d).reshape(DE2)
    return scores[:E2 // 2], scores[E2 // 2:E2]


# speedup vs baseline: 2.6473x; 1.0078x over previous
import functools

import jax
import jax.numpy as jnp
from jax import lax
from jax.experimental import pallas as pl
from jax.experimental.pallas import tpu as pltpu
from jax.experimental.pallas import tpu_sc as plsc

N = 10000
D = 128
E = 320000
E2 = 320000

NC = 2
NS = 16
NW = NC * NS
L = 16

C = 80
CHUNKS = E // C
CPW = CHUNKS // NW
RPT = N // NS

_mesh = functools.partial(
    plsc.VectorSubcoreMesh, core_axis_name="c", subcore_axis_name="s",
    num_cores=NC, num_subcores=NS)

_SC_PARAMS = pltpu.CompilerParams(use_tc_tiling_on_sc=False,
                                  needs_layout_passes=False)


def _tree_sum(vs):
    while len(vs) > 1:
        vs = [a + b for a, b in zip(vs[0::2], vs[1::2])]
    return vs[0]


def _deg_body(dst2d, zeros_hbm, ones_hbm, deg_out, idx_v, ones_v, deg_sh):
    c = lax.axis_index("c")
    s = lax.axis_index("s")
    w = c * NS + s
    pltpu.sync_copy(ones_hbm, ones_v)
    pltpu.sync_copy(dst2d.at[pl.ds(w * CPW, CPW)], idx_v)
    pltpu.sync_copy(zeros_hbm.at[pl.ds(s * RPT, RPT)],
                    deg_sh.at[pl.ds(s * RPT, RPT)])
    plsc.subcore_barrier()

    def body(j, carry):
        pltpu.sync_copy(ones_v, deg_sh.at[idx_v.at[j]], add=True)
        return carry

    lax.fori_loop(0, CPW, body, 0)
    plsc.subcore_barrier()
    pltpu.sync_copy(deg_sh.at[pl.ds(s * RPT, RPT)],
                    deg_out.at[c, pl.ds(s * RPT, RPT)])


def _sc_degree(dst2d, zeros16, ones16):
    k = pl.kernel(
        _deg_body,
        out_type=jax.ShapeDtypeStruct((NC, N, L), jnp.float32),
        mesh=_mesh(),
        compiler_params=_SC_PARAMS,
        scratch_types=[
            pltpu.VMEM((CPW, C), jnp.int32),
            pltpu.VMEM((C, L), jnp.float32),
            pltpu.VMEM_SHARED((N, L), jnp.float32),
        ],
    )
    return k(dst2d, zeros16, ones16)


def _agg_body(g_hbm, src2d, dst2d, out,
              sidx_v, didx_v, r0, r1, acc_sh, gs0, gs1, ss0, ss1):
    c = lax.axis_index("c")
    s = lax.axis_index("s")
    w = c * NS + s
    pltpu.sync_copy(src2d.at[pl.ds(w * CPW, CPW)], sidx_v)
    pltpu.sync_copy(dst2d.at[pl.ds(w * CPW, CPW)], didx_v)
    pltpu.sync_copy(g_hbm.at[pl.ds(s * RPT, RPT)],
                    acc_sh.at[pl.ds(s * RPT, RPT)])
    plsc.subcore_barrier()

    def g_issue(j, r, sem):
        pltpu.async_copy(g_hbm.at[sidx_v.at[j]], r, sem)

    def g_wait(r, sem):
        pltpu.make_async_copy(g_hbm.at[pl.ds(0, C)], r, sem).wait()

    def s_issue(j, r, sem):
        pltpu.async_copy(r, acc_sh.at[didx_v.at[j]], sem, add=True)

    def s_wait(r, sem):
        pltpu.make_async_copy(r, acc_sh.at[pl.ds(0, C)], sem).wait()

    g_issue(0, r0, gs0)
    g_issue(1, r1, gs1)

    def body(j2, carry):
        j = 2 * j2
        g_wait(r0, gs0)
        s_issue(j, r0, ss0)
        s_wait(r0, ss0)

        @pl.when(j + 2 < CPW)
        def _():
            g_issue(j + 2, r0, gs0)

        @pl.when(j + 1 < CPW)
        def _():
            g_wait(r1, gs1)
            s_issue(j + 1, r1, ss1)
            s_wait(r1, ss1)

            @pl.when(j + 3 < CPW)
            def _():
                g_issue(j + 3, r1, gs1)

        return carry

    lax.fori_loop(0, (CPW + 1) // 2, body, 0)
    plsc.subcore_barrier()
    pltpu.sync_copy(acc_sh.at[pl.ds(s * RPT, RPT)],
                    out.at[c, pl.ds(s * RPT, RPT)])


def _sc_aggregate(g, src2d, dst2d):
    k = pl.kernel(
        _agg_body,
        out_type=jax.ShapeDtypeStruct((NC, N, D), jnp.float32),
        mesh=_mesh(),
        compiler_params=_SC_PARAMS,
        scratch_types=[
            pltpu.VMEM((CPW, C), jnp.int32),
            pltpu.VMEM((CPW, C), jnp.int32),
            pltpu.VMEM((C, D), jnp.float32),
            pltpu.VMEM((C, D), jnp.float32),
            pltpu.VMEM_SHARED((N, D), jnp.float32),
            pltpu.SemaphoreType.DMA,
            pltpu.SemaphoreType.DMA,
            pltpu.SemaphoreType.DMA,
            pltpu.SemaphoreType.DMA,
        ],
    )
    return k(g, src2d, dst2d)


DC = 64
DE2 = 321536
DCHUNKS = DE2 // DC
DCPW = DCHUNKS // NW
NBUF = 4


def _dec_body(z_hbm, ab2d, out, idx_v, r0, r1, r2, r3, sc_v, z_sh,
              s0, s1, s2, s3):
    c = lax.axis_index("c")
    s = lax.axis_index("s")
    w = c * NS + s
    pltpu.sync_copy(ab2d.at[pl.ds(w * DCPW, DCPW)], idx_v)
    pltpu.sync_copy(z_hbm.at[pl.ds(s * RPT, RPT)],
                    z_sh.at[pl.ds(s * RPT, RPT)])
    plsc.subcore_barrier()
    bufs = [(r0, s0), (r1, s1), (r2, s2), (r3, s3)]

    def issue(j, r, sem):
        pltpu.async_copy(z_sh.at[idx_v.at[j]], r, sem)

    def wait(r, sem):
        pltpu.make_async_copy(z_hbm.at[pl.ds(0, 2 * DC)], r, sem).wait()

    lane = lax.iota(jnp.int32, L)
    perms = [lane ^ d for d in (8, 4, 2, 1)]

    def compute(j, r):
        def gbody(g0, carry):
            terms = []
            for e in range(L):
                row = g0 * L + e
                prods = []
                for k in range(D // (2 * L)):
                    va = r[row, pl.ds(k * 2 * L, 2 * L)]
                    vb = r[row + DC, pl.ds(k * 2 * L, 2 * L)]
                    a0, a1 = plsc.unpack(va, format=plsc.PackFormat.INTERLEAVED)
                    b0, b1 = plsc.unpack(vb, format=plsc.PackFormat.INTERLEAVED)
                    prods += [a0 * b0, a1 * b1]
                v = _tree_sum(prods)
                for p_ in perms:
                    v = v + v[p_]
                terms.append(jnp.where(lane == e, v, 0.0))
            sc_v[j, pl.ds(g0 * L, L)] = _tree_sum(terms)
            return carry

        lax.fori_loop(0, DC // L, gbody, 0)

    for b in range(NBUF):
        issue(b, *bufs[b])

    def body(j4, carry):
        for b in range(NBUF):
            j = NBUF * j4 + b

            @pl.when(j < DCPW)
            def _():
                wait(*bufs[b])
                compute(j, bufs[b][0])

                @pl.when(j + NBUF < DCPW)
                def _():
                    issue(j + NBUF, *bufs[b])

        return carry

    lax.fori_loop(0, (DCPW + NBUF - 1) // NBUF, body, 0)
    pltpu.sync_copy(sc_v, out.at[pl.ds(w * DCPW, DCPW)])


def _sc_decode(z, ab2d):
    k = pl.kernel(
        _dec_body,
        out_type=jax.ShapeDtypeStruct((DCHUNKS, DC), jnp.float32),
        mesh=_mesh(),
        compiler_params=_SC_PARAMS,
        scratch_types=[
            pltpu.VMEM((DCPW, 2 * DC), jnp.int32),
            pltpu.VMEM((2 * DC, D), jnp.bfloat16),
            pltpu.VMEM((2 * DC, D), jnp.bfloat16),
            pltpu.VMEM((2 * DC, D), jnp.bfloat16),
            pltpu.VMEM((2 * DC, D), jnp.bfloat16),
            pltpu.VMEM((DCPW, DC), jnp.float32),
            pltpu.VMEM_SHARED((N, D), jnp.bfloat16),
            pltpu.SemaphoreType.DMA,
            pltpu.SemaphoreType.DMA,
            pltpu.SemaphoreType.DMA,
            pltpu.SemaphoreType.DMA,
        ],
    )
    return k(z, ab2d)


BLK = 1000
GRID = N // BLK


def _tc_a1_body(x_ref, w1_ref, h_ref):
    h_ref[...] = jnp.dot(x_ref[...], w1_ref[...],
                         preferred_element_type=jnp.float32)


def _tc_a1(x, W1):
    return pl.pallas_call(
        _tc_a1_body,
        grid=(GRID,),
        in_specs=[
            pl.BlockSpec((BLK, D), lambda i: (i, 0)),
            pl.BlockSpec((D, D), lambda i: (0, 0)),
        ],
        out_specs=pl.BlockSpec((BLK, D), lambda i: (i, 0)),
        out_shape=jax.ShapeDtypeStruct((N, D), jnp.float32),
    )(x, W1)


def _tc_a2_body(h_ref, deg_ref, g_ref):
    deg = deg_ref[0] + deg_ref[1]
    dinv = lax.rsqrt(deg + 1.0)[:, :1]
    g_ref[...] = dinv * h_ref[...]


def _tc_a2(h, deg16):
    return pl.pallas_call(
        _tc_a2_body,
        grid=(GRID,),
        in_specs=[
            pl.BlockSpec((BLK, D), lambda i: (i, 0)),
            pl.BlockSpec((NC, BLK, L), lambda i: (0, i, 0)),
        ],
        out_specs=pl.BlockSpec((BLK, D), lambda i: (i, 0)),
        out_shape=jax.ShapeDtypeStruct((N, D), jnp.float32),
    )(h, deg16)


def _tc_b_body(acc_ref, g1_ref, deg_ref, b1_ref, w2_ref, g2_ref):
    deg = deg_ref[0] + deg_ref[1]
    dinv = lax.rsqrt(deg + 1.0)[:, :1]
    z1 = dinv * (acc_ref[0] + acc_ref[1] - g1_ref[...]) + b1_ref[...]
    z1 = jnp.maximum(z1, 0.0)
    h = jnp.dot(z1, w2_ref[...],
                preferred_element_type=jnp.float32)
    g2_ref[...] = dinv * h


def _tc_b(acc1, g1, deg16, b1, W2):
    return pl.pallas_call(
        _tc_b_body,
        grid=(GRID,),
        in_specs=[
            pl.BlockSpec((NC, BLK, D), lambda i: (0, i, 0)),
            pl.BlockSpec((BLK, D), lambda i: (i, 0)),
            pl.BlockSpec((NC, BLK, L), lambda i: (0, i, 0)),
            pl.BlockSpec((1, D), lambda i: (0, 0)),
            pl.BlockSpec((D, D), lambda i: (0, 0)),
        ],
        out_specs=pl.BlockSpec((BLK, D), lambda i: (i, 0)),
        out_shape=jax.ShapeDtypeStruct((N, D), jnp.float32),
    )(acc1, g1, deg16, b1, W2)


def _tc_c_body(acc_ref, g2_ref, deg_ref, b2_ref, z_ref):
    deg = deg_ref[0] + deg_ref[1]
    dinv = lax.rsqrt(deg + 1.0)[:, :1]
    z = dinv * (acc_ref[0] + acc_ref[1] - g2_ref[...]) + b2_ref[...]
    z_ref[...] = z.astype(jnp.bfloat16)


def _tc_c(acc2, g2, deg16, b2):
    return pl.pallas_call(
        _tc_c_body,
        grid=(GRID,),
        in_specs=[
            pl.BlockSpec((NC, BLK, D), lambda i: (0, i, 0)),
            pl.BlockSpec((BLK, D), lambda i: (i, 0)),
            pl.BlockSpec((NC, BLK, L), lambda i: (0, i, 0)),
            pl.BlockSpec((1, D), lambda i: (0, 0)),
        ],
        out_specs=pl.BlockSpec((BLK, D), lambda i: (i, 0)),
        out_shape=jax.ShapeDtypeStruct((N, D), jnp.bfloat16),
    )(acc2, g2, deg16, b2)


def kernel(x, edge_index, pos_edge_index, neg_edge_index, W1, b1, W2, b2):
    src2d = edge_index[0].reshape(CHUNKS, C)
    dst2d = edge_index[1].reshape(CHUNKS, C)
    dec = jnp.concatenate([pos_edge_index, neg_edge_index], axis=1)
    pad = jnp.zeros((2, DE2 - E2), jnp.int32)
    dec = jnp.concatenate([dec, pad], axis=1)
    ab2d = jnp.concatenate([dec[0].reshape(DCHUNKS, DC),
                            dec[1].reshape(DCHUNKS, DC)], axis=1)
    zeros16 = jnp.zeros((N, L), jnp.float32)
    ones16 = jnp.ones((C, L), jnp.float32)

    h1 = _tc_a1(x, W1)
    deg16 = _sc_degree(dst2d, zeros16, ones16)
    g1 = _tc_a2(h1, deg16)
    acc1 = _sc_aggregate(g1, src2d, dst2d)
    g2 = _tc_b(acc1, g1, deg16, b1.reshape(1, D), W2)
    acc2 = _sc_aggregate(g2, src2d, dst2d)
    z = _tc_c(acc2, g2, deg16, b2.reshape(1, D))
    scores = _sc_decode(z, ab2d).reshape(DE2)
    return scores[:E2 // 2], scores[E2 // 2:E2]


# speedup vs baseline: 2.6963x; 1.0185x over previous
import functools

import jax
import jax.numpy as jnp
from jax import lax
from jax.experimental import pallas as pl
from jax.experimental.pallas import tpu as pltpu
from jax.experimental.pallas import tpu_sc as plsc

N = 10000
D = 128
E = 320000
E2 = 320000

NC = 2
NS = 16
NW = NC * NS
L = 16

C = 80
CHUNKS = E // C
CPW = CHUNKS // NW
RPT = N // NS

_mesh = functools.partial(
    plsc.VectorSubcoreMesh, core_axis_name="c", subcore_axis_name="s",
    num_cores=NC, num_subcores=NS)

_SC_PARAMS = pltpu.CompilerParams(use_tc_tiling_on_sc=False,
                                  needs_layout_passes=False)


def _tree_sum(vs):
    while len(vs) > 1:
        vs = [a + b for a, b in zip(vs[0::2], vs[1::2])]
    return vs[0]


def _deg_body(dst2d, zeros_hbm, ones_hbm, deg_out, idx_v, ones_v, deg_sh):
    c = lax.axis_index("c")
    s = lax.axis_index("s")
    w = c * NS + s
    pltpu.sync_copy(ones_hbm, ones_v)
    pltpu.sync_copy(dst2d.at[pl.ds(w * CPW, CPW)], idx_v)
    pltpu.sync_copy(zeros_hbm.at[pl.ds(s * RPT, RPT)],
                    deg_sh.at[pl.ds(s * RPT, RPT)])
    plsc.subcore_barrier()

    def body(j, carry):
        pltpu.sync_copy(ones_v, deg_sh.at[idx_v.at[j]], add=True)
        return carry

    lax.fori_loop(0, CPW, body, 0)
    plsc.subcore_barrier()
    pltpu.sync_copy(deg_sh.at[pl.ds(s * RPT, RPT)],
                    deg_out.at[c, pl.ds(s * RPT, RPT)])


def _sc_degree(dst2d, zeros16, ones16):
    k = pl.kernel(
        _deg_body,
        out_type=jax.ShapeDtypeStruct((NC, N, L), jnp.float32),
        mesh=_mesh(),
        compiler_params=_SC_PARAMS,
        scratch_types=[
            pltpu.VMEM((CPW, C), jnp.int32),
            pltpu.VMEM((C, L), jnp.float32),
            pltpu.VMEM_SHARED((N, L), jnp.float32),
        ],
    )
    return k(dst2d, zeros16, ones16)


def _agg_body(g_hbm, src2d, dst2d, out,
              sidx_v, didx_v, r0, r1, r2, acc_sh,
              gs0, gs1, gs2, ss0, ss1, ss2):
    c = lax.axis_index("c")
    s = lax.axis_index("s")
    w = c * NS + s
    pltpu.sync_copy(src2d.at[pl.ds(w * CPW, CPW)], sidx_v)
    pltpu.sync_copy(dst2d.at[pl.ds(w * CPW, CPW)], didx_v)
    pltpu.sync_copy(g_hbm.at[pl.ds(s * RPT, RPT)],
                    acc_sh.at[pl.ds(s * RPT, RPT)])
    plsc.subcore_barrier()

    def g_issue(j, r, sem):
        pltpu.async_copy(g_hbm.at[sidx_v.at[j]], r, sem)

    def g_wait(r, sem):
        pltpu.make_async_copy(g_hbm.at[pl.ds(0, C)], r, sem).wait()

    def s_issue(j, r, sem):
        pltpu.async_copy(r, acc_sh.at[didx_v.at[j]], sem, add=True)

    def s_wait(r, sem):
        pltpu.make_async_copy(r, acc_sh.at[pl.ds(0, C)], sem).wait()

    rbufs = [r0, r1, r2]
    gsems = [gs0, gs1, gs2]
    ssems = [ss0, ss1, ss2]
    ANB = 3

    for j0 in range(2):
        g_issue(j0, rbufs[j0], gsems[j0])

    def turn(base, b):
        j = base + b

        @pl.when(j < CPW)
        def _():
            g_wait(rbufs[b], gsems[b])
            s_issue(j, rbufs[b], ssems[b])

        @pl.when(jnp.logical_and(j >= 2, j - 2 < CPW))
        def _():
            b2 = (b - 2) % ANB
            s_wait(rbufs[b2], ssems[b2])

        @pl.when(j + 2 < CPW)
        def _():
            b4 = (b + 2) % ANB
            g_issue(j + 2, rbufs[b4], gsems[b4])

    def body(t6, carry):
        base = ANB * t6
        for b in range(ANB):
            turn(base, b)
        return carry

    lax.fori_loop(0, (CPW + 2 + ANB - 1) // ANB + 1, body, 0)
    plsc.subcore_barrier()
    pltpu.sync_copy(acc_sh.at[pl.ds(s * RPT, RPT)],
                    out.at[c, pl.ds(s * RPT, RPT)])


def _sc_aggregate(g, src2d, dst2d):
    k = pl.kernel(
        _agg_body,
        out_type=jax.ShapeDtypeStruct((NC, N, D), jnp.float32),
        mesh=_mesh(),
        compiler_params=_SC_PARAMS,
        scratch_types=[
            pltpu.VMEM((CPW, C), jnp.int32),
            pltpu.VMEM((CPW, C), jnp.int32),
        ] + [pltpu.VMEM((C, D), jnp.float32)] * 3 + [
            pltpu.VMEM_SHARED((N, D), jnp.float32),
        ] + [pltpu.SemaphoreType.DMA] * 6,
    )
    return k(g, src2d, dst2d)


DC = 64
DE2 = 321536
DCHUNKS = DE2 // DC
DCPW = DCHUNKS // NW
NBUF = 4


def _dec_body(z_hbm, ab2d, out, idx_v, r0, r1, r2, r3, sc_v, z_sh,
              s0, s1, s2, s3):
    c = lax.axis_index("c")
    s = lax.axis_index("s")
    w = c * NS + s
    pltpu.sync_copy(ab2d.at[pl.ds(w * DCPW, DCPW)], idx_v)
    pltpu.sync_copy(z_hbm.at[pl.ds(s * RPT, RPT)],
                    z_sh.at[pl.ds(s * RPT, RPT)])
    plsc.subcore_barrier()
    bufs = [(r0, s0), (r1, s1), (r2, s2), (r3, s3)]

    def issue(j, r, sem):
        pltpu.async_copy(z_sh.at[idx_v.at[j]], r, sem)

    def wait(r, sem):
        pltpu.make_async_copy(z_hbm.at[pl.ds(0, 2 * DC)], r, sem).wait()

    lane = lax.iota(jnp.int32, L)
    perms = [lane ^ d for d in (8, 4, 2, 1)]

    def compute(j, r):
        def gbody(g0, carry):
            terms = []
            for e in range(L):
                row = g0 * L + e
                prods = []
                for k in range(D // (2 * L)):
                    va = r[row, pl.ds(k * 2 * L, 2 * L)]
                    vb = r[row + DC, pl.ds(k * 2 * L, 2 * L)]
                    a0, a1 = plsc.unpack(va, format=plsc.PackFormat.INTERLEAVED)
                    b0, b1 = plsc.unpack(vb, format=plsc.PackFormat.INTERLEAVED)
                    prods += [a0 * b0, a1 * b1]
                v = _tree_sum(prods)
                for p_ in perms:
                    v = v + v[p_]
                terms.append(jnp.where(lane == e, v, 0.0))
            sc_v[j, pl.ds(g0 * L, L)] = _tree_sum(terms)
            return carry

        lax.fori_loop(0, DC // L, gbody, 0)

    for b in range(NBUF):
        issue(b, *bufs[b])

    def body(j4, carry):
        for b in range(NBUF):
            j = NBUF * j4 + b

            @pl.when(j < DCPW)
            def _():
                wait(*bufs[b])
                compute(j, bufs[b][0])

                @pl.when(j + NBUF < DCPW)
                def _():
                    issue(j + NBUF, *bufs[b])

        return carry

    lax.fori_loop(0, (DCPW + NBUF - 1) // NBUF, body, 0)
    pltpu.sync_copy(sc_v, out.at[pl.ds(w * DCPW, DCPW)])


def _sc_decode(z, ab2d):
    k = pl.kernel(
        _dec_body,
        out_type=jax.ShapeDtypeStruct((DCHUNKS, DC), jnp.float32),
        mesh=_mesh(),
        compiler_params=_SC_PARAMS,
        scratch_types=[
            pltpu.VMEM((DCPW, 2 * DC), jnp.int32),
            pltpu.VMEM((2 * DC, D), jnp.bfloat16),
            pltpu.VMEM((2 * DC, D), jnp.bfloat16),
            pltpu.VMEM((2 * DC, D), jnp.bfloat16),
            pltpu.VMEM((2 * DC, D), jnp.bfloat16),
            pltpu.VMEM((DCPW, DC), jnp.float32),
            pltpu.VMEM_SHARED((N, D), jnp.bfloat16),
            pltpu.SemaphoreType.DMA,
            pltpu.SemaphoreType.DMA,
            pltpu.SemaphoreType.DMA,
            pltpu.SemaphoreType.DMA,
        ],
    )
    return k(z, ab2d)


BLK = 1000
GRID = N // BLK


def _tc_a1_body(x_ref, w1_ref, h_ref):
    h_ref[...] = jnp.dot(x_ref[...], w1_ref[...],
                         preferred_element_type=jnp.float32)


def _tc_a1(x, W1):
    return pl.pallas_call(
        _tc_a1_body,
        grid=(GRID,),
        in_specs=[
            pl.BlockSpec((BLK, D), lambda i: (i, 0)),
            pl.BlockSpec((D, D), lambda i: (0, 0)),
        ],
        out_specs=pl.BlockSpec((BLK, D), lambda i: (i, 0)),
        out_shape=jax.ShapeDtypeStruct((N, D), jnp.float32),
    )(x, W1)


def _tc_a2_body(h_ref, deg_ref, g_ref):
    deg = deg_ref[0] + deg_ref[1]
    dinv = lax.rsqrt(deg + 1.0)[:, :1]
    g_ref[...] = dinv * h_ref[...]


def _tc_a2(h, deg16):
    return pl.pallas_call(
        _tc_a2_body,
        grid=(GRID,),
        in_specs=[
            pl.BlockSpec((BLK, D), lambda i: (i, 0)),
            pl.BlockSpec((NC, BLK, L), lambda i: (0, i, 0)),
        ],
        out_specs=pl.BlockSpec((BLK, D), lambda i: (i, 0)),
        out_shape=jax.ShapeDtypeStruct((N, D), jnp.float32),
    )(h, deg16)


def _tc_b_body(acc_ref, g1_ref, deg_ref, b1_ref, w2_ref, g2_ref):
    deg = deg_ref[0] + deg_ref[1]
    dinv = lax.rsqrt(deg + 1.0)[:, :1]
    z1 = dinv * (acc_ref[0] + acc_ref[1] - g1_ref[...]) + b1_ref[...]
    z1 = jnp.maximum(z1, 0.0)
    h = jnp.dot(z1, w2_ref[...],
                preferred_element_type=jnp.float32)
    g2_ref[...] = dinv * h


def _tc_b(acc1, g1, deg16, b1, W2):
    return pl.pallas_call(
        _tc_b_body,
        grid=(GRID,),
        in_specs=[
            pl.BlockSpec((NC, BLK, D), lambda i: (0, i, 0)),
            pl.BlockSpec((BLK, D), lambda i: (i, 0)),
            pl.BlockSpec((NC, BLK, L), lambda i: (0, i, 0)),
            pl.BlockSpec((1, D), lambda i: (0, 0)),
            pl.BlockSpec((D, D), lambda i: (0, 0)),
        ],
        out_specs=pl.BlockSpec((BLK, D), lambda i: (i, 0)),
        out_shape=jax.ShapeDtypeStruct((N, D), jnp.float32),
    )(acc1, g1, deg16, b1, W2)


def _tc_c_body(acc_ref, g2_ref, deg_ref, b2_ref, z_ref):
    deg = deg_ref[0] + deg_ref[1]
    dinv = lax.rsqrt(deg + 1.0)[:, :1]
    z = dinv * (acc_ref[0] + acc_ref[1] - g2_ref[...]) + b2_ref[...]
    z_ref[...] = z.astype(jnp.bfloat16)


def _tc_c(acc2, g2, deg16, b2):
    return pl.pallas_call(
        _tc_c_body,
        grid=(GRID,),
        in_specs=[
            pl.BlockSpec((NC, BLK, D), lambda i: (0, i, 0)),
            pl.BlockSpec((BLK, D), lambda i: (i, 0)),
            pl.BlockSpec((NC, BLK, L), lambda i: (0, i, 0)),
            pl.BlockSpec((1, D), lambda i: (0, 0)),
        ],
        out_specs=pl.BlockSpec((BLK, D), lambda i: (i, 0)),
        out_shape=jax.ShapeDtypeStruct((N, D), jnp.bfloat16),
    )(acc2, g2, deg16, b2)


def kernel(x, edge_index, pos_edge_index, neg_edge_index, W1, b1, W2, b2):
    src2d = edge_index[0].reshape(CHUNKS, C)
    dst2d = edge_index[1].reshape(CHUNKS, C)
    dec = jnp.concatenate([pos_edge_index, neg_edge_index], axis=1)
    pad = jnp.zeros((2, DE2 - E2), jnp.int32)
    dec = jnp.concatenate([dec, pad], axis=1)
    ab2d = jnp.concatenate([dec[0].reshape(DCHUNKS, DC),
                            dec[1].reshape(DCHUNKS, DC)], axis=1)
    zeros16 = jnp.zeros((N, L), jnp.float32)
    ones16 = jnp.ones((C, L), jnp.float32)

    h1 = _tc_a1(x, W1)
    deg16 = _sc_degree(dst2d, zeros16, ones16)
    g1 = _tc_a2(h1, deg16)
    acc1 = _sc_aggregate(g1, src2d, dst2d)
    g2 = _tc_b(acc1, g1, deg16, b1.reshape(1, D), W2)
    acc2 = _sc_aggregate(g2, src2d, dst2d)
    z = _tc_c(acc2, g2, deg16, b2.reshape(1, D))
    scores = _sc_decode(z, ab2d).reshape(DE2)
    return scores[:E2 // 2], scores[E2 // 2:E2]
